# baseline (speedup 1.0000x reference)
import functools

import jax
import jax.numpy as jnp
from jax import lax
from jax.experimental import pallas as pl
from jax.experimental.pallas import tpu as pltpu
from jax.experimental.pallas import tpu_sc as plsc

D_MODEL = 64
SEQ = 50
LANES = 16

SEQ_PER_CHUNK = 8
CHUNK = SEQ_PER_CHUNK * SEQ
GATHER = 100
N_GATHER = CHUNK // GATHER


def _pos_encoding(num_words, d_model):
    pos = jnp.arange(num_words, dtype=jnp.float32)[:, None]
    i = jnp.arange(d_model, dtype=jnp.float32)[None, :]
    denom = jnp.power(10000.0, 2.0 * i / d_model)
    angle = pos / denom
    even_mask = (jnp.arange(d_model) % 2 == 0)[None, :]
    return jnp.where(even_mask, jnp.sin(angle), jnp.cos(angle))


def _sc_embed(x_idx, table, pe, n_chunks, chunks_per_w):
    rows_total = n_chunks * CHUNK
    mesh = plsc.VectorSubcoreMesh(core_axis_name="c", subcore_axis_name="s")
    nc = mesh.num_cores

    @functools.partial(
        pl.kernel,
        out_type=jax.ShapeDtypeStruct((rows_total, D_MODEL), jnp.float32),
        mesh=mesh,
        scratch_types=[
            pltpu.VMEM((N_GATHER, GATHER), jnp.int32),
            pltpu.VMEM((CHUNK, D_MODEL), jnp.float32),
            pltpu.VMEM((SEQ, D_MODEL), jnp.float32),
            pltpu.SemaphoreType.DMA,
        ],
        compiler_params=pltpu.CompilerParams(use_tc_tiling_on_sc=False),
    )
    def k(x_hbm, table_hbm, pe_hbm, out_hbm, idx_v, rows_v, pe_v, gsem):
        wid = lax.axis_index("s") * nc + lax.axis_index("c")
        pltpu.sync_copy(pe_hbm, pe_v)

        def chunk_body(i, _):
            c = i * (nc * mesh.num_subcores) + wid
            pltpu.sync_copy(x_hbm.at[c], idx_v)
            handles = [
                pltpu.async_copy(
                    table_hbm.at[idx_v.at[j]],
                    rows_v.at[pl.ds(j * GATHER, GATHER)],
                    gsem,
                )
                for j in range(N_GATHER)
            ]
            for h in handles:
                h.wait()

            def pe_body(s, _):
                pe_vecs = [pe_v[s, pl.ds(q * LANES, LANES)]
                           for q in range(D_MODEL // LANES)]
                for rseq in range(SEQ_PER_CHUNK):
                    r = rseq * SEQ + s
                    for q in range(D_MODEL // LANES):
                        sl = pl.ds(q * LANES, LANES)
                        rows_v[r, sl] = rows_v[r, sl] + pe_vecs[q]
                return 0

            lax.fori_loop(0, SEQ, pe_body, 0)
            pltpu.sync_copy(rows_v, out_hbm.at[pl.ds(c * CHUNK, CHUNK)])
            return 0

        lax.fori_loop(0, chunks_per_w, chunk_body, 0)

    return k(x_idx, table, pe)


def kernel(x, table):
    if x.ndim == 1:
        x = x[None, :]
    batch, seq = x.shape
    d_model = table.shape[1]
    pe = _pos_encoding(seq, d_model).astype(jnp.float32)

    rows_total = batch * seq
    n_chunks = rows_total // CHUNK
    nw = 32
    chunks_per_w = n_chunks // nw

    x_idx = x.reshape(n_chunks, N_GATHER, GATHER)
    out = _sc_embed(x_idx, table, pe, n_chunks, chunks_per_w)
    return out.reshape(batch, seq, d_model)


# speedup vs baseline: 1.0299x; 1.0299x over previous
import functools

import jax
import jax.numpy as jnp
from jax import lax
from jax.experimental import pallas as pl
from jax.experimental.pallas import tpu as pltpu
from jax.experimental.pallas import tpu_sc as plsc

D_MODEL = 64
SEQ = 50
LANES = 16
NW = 32

SEQ_PER_CHUNK = 8
CHUNK = SEQ_PER_CHUNK * SEQ
GATHER = 100
N_GATHER = CHUNK // GATHER


def _pos_encoding(num_words, d_model):
    pos = jnp.arange(num_words, dtype=jnp.float32)[:, None]
    i = jnp.arange(d_model, dtype=jnp.float32)[None, :]
    denom = jnp.power(10000.0, 2.0 * i / d_model)
    angle = pos / denom
    even_mask = (jnp.arange(d_model) % 2 == 0)[None, :]
    return jnp.where(even_mask, jnp.sin(angle), jnp.cos(angle))


def _sc_embed(x_idx, table, pe, n_chunks, chunks_per_w):
    rows_total = n_chunks * CHUNK
    mesh = plsc.VectorSubcoreMesh(core_axis_name="c", subcore_axis_name="s")
    nc = mesh.num_cores

    @functools.partial(
        pl.kernel,
        out_type=jax.ShapeDtypeStruct((rows_total, D_MODEL), jnp.float32),
        mesh=mesh,
        scratch_types=[
            pltpu.VMEM((2, N_GATHER, GATHER), jnp.int32),
            pltpu.VMEM((2, CHUNK, D_MODEL), jnp.float32),
            pltpu.VMEM((SEQ, D_MODEL), jnp.float32),
            [pltpu.SemaphoreType.DMA] * 2,
            [pltpu.SemaphoreType.DMA] * 2,
        ],
        compiler_params=pltpu.CompilerParams(use_tc_tiling_on_sc=False),
    )
    def k(x_hbm, table_hbm, pe_hbm, out_hbm, idx_v, rows_v, pe_v, gsem, ssem):
        wid = lax.axis_index("s") * nc + lax.axis_index("c")
        pltpu.sync_copy(pe_hbm, pe_v)

        def chunk_of(kk):
            return kk * NW + wid

        def fire_gathers(kk, p):
            pltpu.sync_copy(x_hbm.at[chunk_of(kk)], idx_v.at[p])
            for j in range(N_GATHER):
                pltpu.async_copy(
                    table_hbm.at[idx_v.at[p].at[j]],
                    rows_v.at[p].at[pl.ds(j * GATHER, GATHER)],
                    gsem[p],
                )

        def drain_gathers(p):
            pltpu.make_async_copy(
                out_hbm.at[pl.ds(0, CHUNK)], rows_v.at[p], gsem[p]
            ).wait()

        def drain_store(p):
            pltpu.make_async_copy(
                out_hbm.at[pl.ds(0, CHUNK)], rows_v.at[p], ssem[p]
            ).wait()

        def add_pe_and_store(kk, p):
            rp = rows_v.at[p]

            def pe_body(s, _):
                pe_vecs = [pe_v[s, pl.ds(q * LANES, LANES)]
                           for q in range(D_MODEL // LANES)]
                for rseq in range(SEQ_PER_CHUNK):
                    r = rseq * SEQ + s
                    for q in range(D_MODEL // LANES):
                        sl = pl.ds(q * LANES, LANES)
                        rp[r, sl] = rp[r, sl] + pe_vecs[q]
                return 0

            lax.fori_loop(0, SEQ, pe_body, 0)
            pltpu.async_copy(
                rows_v.at[p], out_hbm.at[pl.ds(chunk_of(kk) * CHUNK, CHUNK)],
                ssem[p],
            )

        fire_gathers(0, 0)

        drain_gathers(0)
        fire_gathers(1, 1)
        add_pe_and_store(0, 0)

        @pl.loop(1, chunks_per_w - 1, step=2)
        def body(t):
            for b in range(2):
                kk = t + b
                p = (1 + b) % 2
                drain_gathers(p)
                drain_store(1 - p)
                fire_gathers(kk + 1, 1 - p)
                add_pe_and_store(kk, p)

        p_last = (chunks_per_w - 1) % 2
        drain_gathers(p_last)
        drain_store(1 - p_last)
        add_pe_and_store(chunks_per_w - 1, p_last)
        drain_store(p_last)

    return k(x_idx, table, pe)


def kernel(x, table):
    if x.ndim == 1:
        x = x[None, :]
    batch, seq = x.shape
    d_model = table.shape[1]
    pe = _pos_encoding(seq, d_model).astype(jnp.float32)

    rows_total = batch * seq
    n_chunks = rows_total // CHUNK
    chunks_per_w = n_chunks // NW

    x_idx = x.reshape(n_chunks, N_GATHER, GATHER)
    out = _sc_embed(x_idx, table, pe, n_chunks, chunks_per_w)
    return out.reshape(batch, seq, d_model)
